# baseline (speedup 1.0000x reference)
import functools

import jax
import jax.numpy as jnp
from jax import lax
from jax.experimental import pallas as pl
from jax.experimental.pallas import tpu as pltpu
from jax.experimental.pallas import tpu_sc as plsc

E = 64
K = 8
N_GROUP = 8
TOPK_GROUP = 4
EPG = E // N_GROUP
SCALE = 2.5

NC, NS, L = 2, 16, 16
NW = NC * NS


def _tc_scores(x, weight, bias_col, n_tok, tpw, tile=None):
    if tile is None:
        tile = tpw
    grid = n_tok // tile
    h = x.shape[1]

    sub = tile // tpw

    def body(x_ref, w_ref, b_ref, o_ref):
        for k in range(sub):
            logits = lax.dot_general(
                w_ref[...], x_ref[pl.ds(k * tpw, tpw), :],
                dimension_numbers=(((1,), (1,)), ((), ())),
                preferred_element_type=jnp.float32,
            )
            o_ref[k] = jax.nn.sigmoid(logits) + b_ref[...]

    return pl.pallas_call(
        body,
        grid=(grid,),
        in_specs=[
            pl.BlockSpec((tile, h), lambda i: (i, 0)),
            pl.BlockSpec((E, h), lambda i: (0, 0)),
            pl.BlockSpec((E, 1), lambda i: (0, 0)),
        ],
        out_specs=pl.BlockSpec((sub, E, tpw), lambda i: (i, 0, 0)),
        out_shape=jax.ShapeDtypeStruct((n_tok // tpw, E, tpw), jnp.float32),
    )(x, weight, bias_col)


CHUNK = 128


def _sc_route(sfc_slabs, bias, n_tok, tpw):
    n_chunk = tpw // CHUNK
    n_slab = CHUNK // L
    mesh = plsc.VectorSubcoreMesh(
        core_axis_name="c", subcore_axis_name="s",
        num_cores=NC, num_subcores=NS,
    )

    @functools.partial(
        pl.kernel,
        out_type=(
            jax.ShapeDtypeStruct((n_tok, K), jnp.int32),
            jax.ShapeDtypeStruct((n_tok, K), jnp.float32),
        ),
        mesh=mesh,
        compiler_params=pltpu.CompilerParams(needs_layout_passes=False),
        scratch_types=[
            pltpu.VMEM((2, E, CHUNK), jnp.float32),
            pltpu.VMEM((E,), jnp.float32),
            pltpu.VMEM((2, CHUNK, K), jnp.int32),
            pltpu.VMEM((2, CHUNK, K), jnp.float32),
            pltpu.SemaphoreType.DMA,
            pltpu.SemaphoreType.DMA,
            pltpu.SemaphoreType.DMA,
            pltpu.SemaphoreType.DMA,
        ],
    )
    def route(sfc_hbm, bias_hbm, idx_hbm, wgt_hbm, bufs, biasv, idxb, wgtb,
              sin0, sin1, sout0, sout1):
        wid = lax.axis_index("s") * NC + lax.axis_index("c")
        pltpu.sync_copy(bias_hbm, biasv)
        lanes = lax.iota(jnp.int32, L)
        neg1 = jnp.full((L,), -1.0, jnp.float32)
        sems_in = [sin0, sin1]
        sems_out = [sout0, sout1]

        def make_slab_body(buf, idxc, wgtc):
            def slab_body(s, carry):
                tok = s * L + lanes

                gs_l, gm_l, gmi_l = [], [], []
                for g in range(N_GROUP):
                    m1 = buf[EPG * g, pl.ds(s * L, L)]
                    m2 = neg1
                    mi = jnp.full((L,), EPG * g, jnp.int32)
                    for j in range(1, EPG):
                        x = buf[EPG * g + j, pl.ds(s * L, L)]
                        m2 = jnp.maximum(m2, jnp.minimum(m1, x))
                        gt = x > m1
                        mi = jnp.where(gt, EPG * g + j, mi)
                        m1 = jnp.maximum(m1, x)
                    gs_l.append(m1 + m2)
                    gm_l.append(m1)
                    gmi_l.append(mi)

                for g in range(N_GROUP):
                    cnt = jnp.zeros((L,), jnp.int32)
                    for j in range(N_GROUP):
                        if j == g:
                            continue
                        beats = (gs_l[j] >= gs_l[g]) if j < g else (gs_l[j] > gs_l[g])
                        cnt = cnt + beats.astype(jnp.int32)
                    gm_l[g] = jnp.where(cnt < TOPK_GROUP, gm_l[g], neg1)

                wsum = jnp.zeros((L,), jnp.float32)
                wvals = []
                for r in range(K):
                    m = gm_l[0]
                    mi = gmi_l[0]
                    gw = jnp.zeros((L,), jnp.int32)
                    for g in range(1, N_GROUP):
                        gt = gm_l[g] > m
                        mi = jnp.where(gt, gmi_l[g], mi)
                        gw = jnp.where(gt, g, gw)
                        m = jnp.maximum(m, gm_l[g])
                    wr = m - plsc.load_gather(biasv, [mi])
                    wvals.append(wr)
                    wsum = wsum + wr
                    plsc.store_scatter(idxc, [tok, jnp.full((L,), r, jnp.int32)], mi)
                    if r == K - 1:
                        break
                    plsc.store_scatter(buf, [mi, tok], neg1)
                    base = gw * EPG
                    m1 = neg1
                    mi1 = base
                    for j in range(EPG):
                        ev = base + j
                        x = plsc.load_gather(buf, [ev, tok])
                        gt = x > m1
                        mi1 = jnp.where(gt, ev, mi1)
                        m1 = jnp.maximum(m1, x)
                    for g in range(N_GROUP):
                        upd = gw == g
                        gm_l[g] = jnp.where(upd, m1, gm_l[g])
                        gmi_l[g] = jnp.where(upd, mi1, gmi_l[g])

                denom = wsum + 1e-20
                for r in range(K):
                    wn = (wvals[r] / denom) * SCALE
                    plsc.store_scatter(wgtc, [tok, jnp.full((L,), r, jnp.int32)], wn)
                return carry
            return slab_body

        in_copies = [
            pltpu.make_async_copy(
                sfc_hbm.at[wid, :, pl.ds(c * CHUNK, CHUNK)],
                bufs.at[c % 2], sems_in[c % 2])
            for c in range(n_chunk)
        ]
        out_copies = [None] * n_chunk
        in_copies[0].start()
        for c in range(n_chunk):
            if c + 1 < n_chunk:
                in_copies[c + 1].start()
            cb = c % 2
            in_copies[c].wait()
            if c >= 2:
                for cp in out_copies[c - 2]:
                    cp.wait()
            lax.fori_loop(0, n_slab,
                          make_slab_body(bufs.at[cb], idxb.at[cb], wgtb.at[cb]), 0)
            base = wid * tpw + c * CHUNK
            out_copies[c] = (
                pltpu.make_async_copy(
                    idxb.at[cb], idx_hbm.at[pl.ds(base, CHUNK), :], sems_out[cb]),
                pltpu.make_async_copy(
                    wgtb.at[cb], wgt_hbm.at[pl.ds(base, CHUNK), :], sems_out[cb]),
            )
            for cp in out_copies[c]:
                cp.start()
        for c in range(max(0, n_chunk - 2), n_chunk):
            for cp in out_copies[c]:
                cp.wait()

    return route(sfc_slabs, bias)


N_SUPER = 4


def kernel(hidden_states, weight, e_score_correction_bias):
    b, s, h = hidden_states.shape
    n_tok = b * s
    x = hidden_states.reshape(n_tok, h).astype(jnp.float32)
    w = weight.astype(jnp.float32)
    bias = e_score_correction_bias.astype(jnp.float32)
    n_sub = n_tok // N_SUPER
    tpw = n_sub // NW
    idxs, wgts = [], []
    for c in range(N_SUPER):
        xc = lax.slice_in_dim(x, c * n_sub, (c + 1) * n_sub, axis=0)
        sfc_c = _tc_scores(xc, w, bias[:, None], n_sub, tpw, tile=2048)
        idx_c, wgt_c = _sc_route(sfc_c, bias, n_sub, tpw)
        idxs.append(idx_c)
        wgts.append(wgt_c)
    return jnp.concatenate(idxs, axis=0), jnp.concatenate(wgts, axis=0)


# speedup vs baseline: 1.7290x; 1.7290x over previous
import functools

import jax
import jax.numpy as jnp
from jax import lax
from jax.experimental import pallas as pl
from jax.experimental.pallas import tpu as pltpu
from jax.experimental.pallas import tpu_sc as plsc

E = 64
K = 8
N_GROUP = 8
TOPK_GROUP = 4
EPG = E // N_GROUP
SCALE = 2.5

NC, NS, L = 2, 16, 16
NW = NC * NS


def _tc_scores(x, weight, bias_col, n_tok, tpw, tile=None):
    if tile is None:
        tile = tpw
    grid = n_tok // tile
    h = x.shape[1]

    sub = tile // tpw

    def body(x_ref, w_ref, b_ref, o_ref):
        for k in range(sub):
            logits = lax.dot_general(
                w_ref[...], x_ref[pl.ds(k * tpw, tpw), :],
                dimension_numbers=(((1,), (1,)), ((), ())),
                preferred_element_type=jnp.float32,
            )
            o_ref[k] = jax.nn.sigmoid(logits) + b_ref[...]

    return pl.pallas_call(
        body,
        grid=(grid,),
        in_specs=[
            pl.BlockSpec((tile, h), lambda i: (i, 0)),
            pl.BlockSpec((E, h), lambda i: (0, 0)),
            pl.BlockSpec((E, 1), lambda i: (0, 0)),
        ],
        out_specs=pl.BlockSpec((sub, E, tpw), lambda i: (i, 0, 0)),
        out_shape=jax.ShapeDtypeStruct((n_tok // tpw, E, tpw), jnp.float32),
    )(x, weight, bias_col)


CHUNK = 128


def _sc_route(sfc_slabs, bias, n_tok, tpw):
    n_chunk = tpw // CHUNK
    n_slab = CHUNK // L
    mesh = plsc.VectorSubcoreMesh(
        core_axis_name="c", subcore_axis_name="s",
        num_cores=NC, num_subcores=NS,
    )

    @functools.partial(
        pl.kernel,
        out_type=(
            jax.ShapeDtypeStruct((n_tok, K), jnp.int32),
            jax.ShapeDtypeStruct((n_tok, K), jnp.float32),
        ),
        mesh=mesh,
        compiler_params=pltpu.CompilerParams(needs_layout_passes=False),
        scratch_types=[
            pltpu.VMEM((2, E, CHUNK), jnp.float32),
            pltpu.VMEM((E,), jnp.float32),
            pltpu.VMEM((2, CHUNK, K), jnp.int32),
            pltpu.VMEM((2, CHUNK, K), jnp.float32),
            pltpu.SemaphoreType.DMA,
            pltpu.SemaphoreType.DMA,
            pltpu.SemaphoreType.DMA,
            pltpu.SemaphoreType.DMA,
        ],
    )
    def route(sfc_hbm, bias_hbm, idx_hbm, wgt_hbm, bufs, biasv, idxb, wgtb,
              sin0, sin1, sout0, sout1):
        wid = lax.axis_index("s") * NC + lax.axis_index("c")
        pltpu.sync_copy(bias_hbm, biasv)
        lanes = lax.iota(jnp.int32, L)
        neg1 = jnp.full((L,), -1.0, jnp.float32)
        sems_in = [sin0, sin1]
        sems_out = [sout0, sout1]

        def make_slab_body(buf, idxc, wgtc):
            def slab_body(s, carry):
                tok = s * L + lanes

                gs_l, gm_l, gmi_l = [], [], []
                for g in range(N_GROUP):
                    m1 = buf[EPG * g, pl.ds(s * L, L)]
                    m2 = neg1
                    mi = jnp.full((L,), EPG * g, jnp.int32)
                    for j in range(1, EPG):
                        x = buf[EPG * g + j, pl.ds(s * L, L)]
                        m2 = jnp.maximum(m2, jnp.minimum(m1, x))
                        gt = x > m1
                        mi = jnp.where(gt, EPG * g + j, mi)
                        m1 = jnp.maximum(m1, x)
                    gs_l.append(m1 + m2)
                    gm_l.append(m1)
                    gmi_l.append(mi)

                for g in range(N_GROUP):
                    cnt = jnp.zeros((L,), jnp.int32)
                    for j in range(N_GROUP):
                        if j == g:
                            continue
                        beats = (gs_l[j] >= gs_l[g]) if j < g else (gs_l[j] > gs_l[g])
                        cnt = cnt + beats.astype(jnp.int32)
                    gm_l[g] = jnp.where(cnt < TOPK_GROUP, gm_l[g], neg1)

                wsum = jnp.zeros((L,), jnp.float32)
                wvals = []
                for r in range(K):
                    m = gm_l[0]
                    mi = gmi_l[0]
                    gw = jnp.zeros((L,), jnp.int32)
                    for g in range(1, N_GROUP):
                        gt = gm_l[g] > m
                        mi = jnp.where(gt, gmi_l[g], mi)
                        gw = jnp.where(gt, g, gw)
                        m = jnp.maximum(m, gm_l[g])
                    wr = m - plsc.load_gather(biasv, [mi])
                    wvals.append(wr)
                    wsum = wsum + wr
                    plsc.store_scatter(idxc, [tok, jnp.full((L,), r, jnp.int32)], mi)
                    if r == K - 1:
                        break
                    plsc.store_scatter(buf, [mi, tok], neg1)
                    base = gw * EPG
                    m1 = neg1
                    mi1 = base
                    for j in range(EPG):
                        ev = base + j
                        x = plsc.load_gather(buf, [ev, tok])
                        gt = x > m1
                        mi1 = jnp.where(gt, ev, mi1)
                        m1 = jnp.maximum(m1, x)
                    for g in range(N_GROUP):
                        upd = gw == g
                        gm_l[g] = jnp.where(upd, m1, gm_l[g])
                        gmi_l[g] = jnp.where(upd, mi1, gmi_l[g])

                denom = wsum + 1e-20
                for r in range(K):
                    wn = (wvals[r] / denom) * SCALE
                    plsc.store_scatter(wgtc, [tok, jnp.full((L,), r, jnp.int32)], wn)
                return carry
            return slab_body

        in_copies = [
            pltpu.make_async_copy(
                sfc_hbm.at[wid, :, pl.ds(c * CHUNK, CHUNK)],
                bufs.at[c % 2], sems_in[c % 2])
            for c in range(n_chunk)
        ]
        out_copies = [None] * n_chunk
        in_copies[0].start()
        for c in range(n_chunk):
            if c + 1 < n_chunk:
                in_copies[c + 1].start()
            cb = c % 2
            in_copies[c].wait()
            if c >= 2:
                for cp in out_copies[c - 2]:
                    cp.wait()
            lax.fori_loop(0, n_slab,
                          make_slab_body(bufs.at[cb], idxb.at[cb], wgtb.at[cb]), 0)
            base = wid * tpw + c * CHUNK
            out_copies[c] = (
                pltpu.make_async_copy(
                    idxb.at[cb], idx_hbm.at[pl.ds(base, CHUNK), :], sems_out[cb]),
                pltpu.make_async_copy(
                    wgtb.at[cb], wgt_hbm.at[pl.ds(base, CHUNK), :], sems_out[cb]),
            )
            for cp in out_copies[c]:
                cp.start()
        for c in range(max(0, n_chunk - 2), n_chunk):
            for cp in out_copies[c]:
                cp.wait()

    return route(sfc_slabs, bias)


N_SUPER = 1


def kernel(hidden_states, weight, e_score_correction_bias):
    b, s, h = hidden_states.shape
    n_tok = b * s
    x = hidden_states.reshape(n_tok, h).astype(jnp.float32)
    w = weight.astype(jnp.float32)
    bias = e_score_correction_bias.astype(jnp.float32)
    n_sub = n_tok // N_SUPER
    tpw = n_sub // NW
    idxs, wgts = [], []
    for c in range(N_SUPER):
        xc = lax.slice_in_dim(x, c * n_sub, (c + 1) * n_sub, axis=0)
        sfc_c = _tc_scores(xc, w, bias[:, None], n_sub, tpw, tile=2048)
        idx_c, wgt_c = _sc_route(sfc_c, bias, n_sub, tpw)
        idxs.append(idx_c)
        wgts.append(wgt_c)
    return jnp.concatenate(idxs, axis=0), jnp.concatenate(wgts, axis=0)
